# baseline (speedup 1.0000x reference)
import functools

import jax
import jax.numpy as jnp
import numpy as np
from jax import lax
from jax.experimental import pallas as pl
from jax.experimental.pallas import tpu as pltpu
from jax.experimental.pallas import tpu_sc as plsc

_N = 1048576
_NUM_CORES = 2
_NUM_SUBCORES = 16
_LANES = 16
_NUM_WORKERS = _NUM_CORES * _NUM_SUBCORES

_MAIN = 30720
_CHUNKS = 2
_CHUNK = _MAIN // _CHUNKS
_TAIL = 4096
_TAIL_BASE = _MAIN * _NUM_WORKERS

_BINS = np.arange(-8.0, 8.0, 0.25).astype(np.float32)
_T = (1.0 / (1.0 + np.exp(-_BINS.astype(np.float64)))).astype(np.float32)
_TLO = _T.copy()
_TLO[0] = 0.0
_J = np.clip(np.arange(80) - 5, 0, 63)
_TLO_EXT = _TLO[_J].astype(np.float32)
_OUT_EXT = _BINS[_J].astype(np.float32)

_SCALE = np.float32(4.0 * np.log(2.0) / 2.0**23)
_BIAS = np.float32(32.0 + 0.25 + 5.0)

_UNROLL = 8


def _body(x_hbm, tlo_hbm, out_tab_hbm, out_hbm, xv, ov, tlov, otv, insem,
          outsem):
    s = lax.axis_index("s")
    c = lax.axis_index("c")
    wid = s * _NUM_CORES + c
    base = wid * _MAIN
    tail_base = _TAIL_BASE + s * _TAIL

    in_copies = [
        pltpu.async_copy(
            x_hbm.at[pl.ds(base + k * _CHUNK, _CHUNK)],
            xv.at[pl.ds(k * _CHUNK, _CHUNK)],
            insem.at[k],
        )
        for k in range(_CHUNKS)
    ]

    @pl.when(c == 1)
    def _issue_tail():
        pltpu.async_copy(
            x_hbm.at[pl.ds(tail_base, _TAIL)],
            xv.at[pl.ds(_MAIN, _TAIL)],
            insem.at[_CHUNKS],
        )

    pltpu.sync_copy(tlo_hbm, tlov)
    pltpu.sync_copy(out_tab_hbm, otv)

    def compute(lo, hi):
        @plsc.parallel_loop(lo, hi, step=_LANES, unroll=_UNROLL)
        def _(off):
            x = xv[pl.ds(off, _LANES)]
            bx = lax.bitcast_convert_type(x, jnp.int32)
            bt = lax.bitcast_convert_type(1.0 - x, jnp.int32)
            d = (bx - bt).astype(jnp.float32)
            gidx = (d * _SCALE + _BIAS).astype(jnp.int32)
            tlo = plsc.load_gather(tlov, [gidx])
            gidx = jnp.where(x < tlo, gidx - 1, gidx)
            ov[pl.ds(off, _LANES)] = plsc.load_gather(otv, [gidx])

    for k in range(_CHUNKS):
        in_copies[k].wait()
        compute(k * _CHUNK, (k + 1) * _CHUNK)
        pltpu.async_copy(
            ov.at[pl.ds(k * _CHUNK, _CHUNK)],
            out_hbm.at[pl.ds(base + k * _CHUNK, _CHUNK)],
            outsem.at[k],
        )

    @pl.when(c == 1)
    def _tail_work():
        pltpu.make_async_copy(
            x_hbm.at[pl.ds(tail_base, _TAIL)],
            xv.at[pl.ds(_MAIN, _TAIL)],
            insem.at[_CHUNKS],
        ).wait()
        compute(_MAIN, _MAIN + _TAIL)
        pltpu.async_copy(
            ov.at[pl.ds(_MAIN, _TAIL)],
            out_hbm.at[pl.ds(tail_base, _TAIL)],
            outsem.at[_CHUNKS],
        )

    for k in range(_CHUNKS):
        pltpu.make_async_copy(
            ov.at[pl.ds(k * _CHUNK, _CHUNK)],
            out_hbm.at[pl.ds(base + k * _CHUNK, _CHUNK)],
            outsem.at[k],
        ).wait()

    @pl.when(c == 1)
    def _drain_tail():
        pltpu.make_async_copy(
            ov.at[pl.ds(_MAIN, _TAIL)],
            out_hbm.at[pl.ds(tail_base, _TAIL)],
            outsem.at[_CHUNKS],
        ).wait()


_sc_call = functools.partial(
    pl.kernel,
    out_type=jax.ShapeDtypeStruct((_N,), jnp.float32),
    mesh=plsc.VectorSubcoreMesh(core_axis_name="c", subcore_axis_name="s"),
    compiler_params=pltpu.CompilerParams(needs_layout_passes=False),
    scratch_types=[
        pltpu.VMEM((_MAIN + _TAIL,), jnp.float32),
        pltpu.VMEM((_MAIN + _TAIL,), jnp.float32),
        pltpu.VMEM((80,), jnp.float32),
        pltpu.VMEM((80,), jnp.float32),
        pltpu.SemaphoreType.DMA((_CHUNKS + 1,)),
        pltpu.SemaphoreType.DMA((_CHUNKS + 1,)),
    ],
)(_body)


@jax.jit
def kernel(Xs):
    return _sc_call(Xs, jnp.asarray(_TLO_EXT), jnp.asarray(_OUT_EXT))


# speedup vs baseline: 1.0071x; 1.0071x over previous
import functools

import jax
import jax.numpy as jnp
import numpy as np
from jax import lax
from jax.experimental import pallas as pl
from jax.experimental.pallas import tpu as pltpu
from jax.experimental.pallas import tpu_sc as plsc

_N = 1048576
_NUM_CORES = 2
_NUM_SUBCORES = 16
_LANES = 16
_NUM_WORKERS = _NUM_CORES * _NUM_SUBCORES
_PER_WORKER = _N // _NUM_WORKERS
_CHUNKS = 2
_CHUNK = _PER_WORKER // _CHUNKS

_BINS = np.arange(-8.0, 8.0, 0.25).astype(np.float32)
_T = (1.0 / (1.0 + np.exp(-_BINS.astype(np.float64)))).astype(np.float32)
_TLO = _T.copy()
_TLO[0] = 0.0
_J = np.clip(np.arange(80) - 5, 0, 63)
_TLO_EXT = _TLO[_J].astype(np.float32)
_OUT_EXT = _BINS[_J].astype(np.float32)

_SCALE = np.float32(4.0 * np.log(2.0) / 2.0**23)
_BIAS = np.float32(32.0 + 0.25 + 5.0)

_UNROLL = 8


def _body(x_hbm, tlo_hbm, out_tab_hbm, out_hbm, xv, ov, tlov, otv, insem,
          outsem):
    wid = lax.axis_index("s") * _NUM_CORES + lax.axis_index("c")
    base = wid * _PER_WORKER

    in_copies = [
        pltpu.async_copy(
            x_hbm.at[pl.ds(base + c * _CHUNK, _CHUNK)],
            xv.at[pl.ds(c * _CHUNK, _CHUNK)],
            insem.at[c],
        )
        for c in range(_CHUNKS)
    ]
    pltpu.sync_copy(tlo_hbm, tlov)
    pltpu.sync_copy(out_tab_hbm, otv)

    out_copies = []
    for c in range(_CHUNKS):
        in_copies[c].wait()

        @plsc.parallel_loop(c * _CHUNK, (c + 1) * _CHUNK, step=_LANES,
                            unroll=_UNROLL)
        def _(off):
            x = xv[pl.ds(off, _LANES)]
            bx = lax.bitcast_convert_type(x, jnp.int32)
            bt = lax.bitcast_convert_type(1.0 - x, jnp.int32)
            d = (bx - bt).astype(jnp.float32)
            gidx = (d * _SCALE + _BIAS).astype(jnp.int32)
            tlo = plsc.load_gather(tlov, [gidx])
            gidx = gidx - (x < tlo).astype(jnp.int32)
            ov[pl.ds(off, _LANES)] = plsc.load_gather(otv, [gidx])

        out_copies.append(
            pltpu.async_copy(
                ov.at[pl.ds(c * _CHUNK, _CHUNK)],
                out_hbm.at[pl.ds(base + c * _CHUNK, _CHUNK)],
                outsem.at[c],
            ))
    for d in out_copies:
        d.wait()


_sc_call = functools.partial(
    pl.kernel,
    out_type=jax.ShapeDtypeStruct((_N,), jnp.float32),
    mesh=plsc.VectorSubcoreMesh(core_axis_name="c", subcore_axis_name="s"),
    compiler_params=pltpu.CompilerParams(needs_layout_passes=False),
    scratch_types=[
        pltpu.VMEM((_PER_WORKER,), jnp.float32),
        pltpu.VMEM((_PER_WORKER,), jnp.float32),
        pltpu.VMEM((80,), jnp.float32),
        pltpu.VMEM((80,), jnp.float32),
        pltpu.SemaphoreType.DMA((_CHUNKS,)),
        pltpu.SemaphoreType.DMA((_CHUNKS,)),
    ],
)(_body)


@jax.jit
def kernel(Xs):
    return _sc_call(Xs, jnp.asarray(_TLO_EXT), jnp.asarray(_OUT_EXT))
